# baseline (speedup 1.0000x reference)
import functools

import jax
import jax.numpy as jnp
from jax import lax
from jax.experimental import pallas as pl
from jax.experimental.pallas import tpu as pltpu
from jax.experimental.pallas import tpu_sc as plsc

N_HASHES = 8
BLK = 128
R = 1.0

_NC = 2
_NS = 16
_NW = _NC * _NS


def _sc_gather_body(per_w, chunk, table_hbm, idx2_hbm, out_hbm,
                    idx_all, rows0, rows1, sw0, sw1, sg0, sg1):
    wid = lax.axis_index("s") * _NC + lax.axis_index("c")
    base = wid * per_w
    nst = per_w // chunk
    pltpu.sync_copy(idx2_hbm.at[pl.ds(base, per_w)], idx_all)

    def step(i, carry):
        off0 = base + (2 * i) * chunk
        off1 = off0 + chunk

        @pl.when(i > 0)
        def _():
            pltpu.make_async_copy(
                rows0, out_hbm.at[pl.ds(base, chunk)], sw0).wait()

        pltpu.async_copy(
            table_hbm.at[idx_all.at[pl.ds((2 * i) * chunk, chunk)]],
            rows0, sg0).wait()
        pltpu.async_copy(rows0, out_hbm.at[pl.ds(off0, chunk)], sw0)

        @pl.when(i > 0)
        def _():
            pltpu.make_async_copy(
                rows1, out_hbm.at[pl.ds(base, chunk)], sw1).wait()

        pltpu.async_copy(
            table_hbm.at[idx_all.at[pl.ds((2 * i + 1) * chunk, chunk)]],
            rows1, sg1).wait()
        pltpu.async_copy(rows1, out_hbm.at[pl.ds(off1, chunk)], sw1)
        return carry

    lax.fori_loop(0, nst // 2, step, 0)
    pltpu.make_async_copy(rows0, out_hbm.at[pl.ds(base, chunk)], sw0).wait()
    pltpu.make_async_copy(rows1, out_hbm.at[pl.ds(base, chunk)], sw1).wait()


def _sc_gather(table, idx, chunk=256):
    n = idx.shape[0]
    d = table.shape[1]
    per_w = n // _NW
    mesh = plsc.VectorSubcoreMesh(core_axis_name="c", subcore_axis_name="s")
    f = pl.kernel(
        functools.partial(_sc_gather_body, per_w, chunk),
        out_type=jax.ShapeDtypeStruct((n, d), table.dtype),
        mesh=mesh,
        scratch_types=[
            pltpu.VMEM((per_w,), jnp.int32),
            pltpu.VMEM((chunk, d), table.dtype),
            pltpu.VMEM((chunk, d), table.dtype),
            pltpu.SemaphoreType.DMA,
            pltpu.SemaphoreType.DMA,
            pltpu.SemaphoreType.DMA,
            pltpu.SemaphoreType.DMA,
        ],
    )
    return f(table, idx)


def _attn_body(q_ref, kv_ref, o_ref):
    q = q_ref[:, :, :64]
    k = kv_ref[:, :, :64]
    v = kv_ref[:, :, 64:]
    inner = jax.lax.dot_general(
        q, k, (((2,), (2,)), ((0,), (0,))), preferred_element_type=jnp.float32)
    m = jnp.max(inner, axis=-1, keepdims=True)
    e = jnp.exp(inner - m)
    s = jnp.sum(e, axis=-1, keepdims=True)
    o = jax.lax.dot_general(
        e, v, (((2,), (1,)), ((0,), (0,))), preferred_element_type=jnp.float32)
    lse = jnp.log(s) + m
    o_ref[...] = jnp.concatenate(
        [o / s, jnp.broadcast_to(lse, o.shape)], axis=-1)


def _block_attention(s_all, g=8):
    nb = s_all.shape[0] // 2
    kv_off = nb // g
    return pl.pallas_call(
        _attn_body,
        grid=(nb // g,),
        in_specs=[
            pl.BlockSpec((g, BLK, 128), lambda i: (i, 0, 0)),
            pl.BlockSpec((g, BLK, 128), lambda i: (i + kv_off, 0, 0)),
        ],
        out_specs=pl.BlockSpec((g, BLK, 128), lambda i: (i, 0, 0)),
        out_shape=jax.ShapeDtypeStruct((nb, BLK, 128), jnp.float32),
    )(s_all, s_all)


def _combine_body(*refs):
    out_ref = refs[-1]
    oa = jnp.concatenate([r[...] for r in refs[:-1]], axis=0)
    o = oa[:, :, :64]
    logits = oa[:, :, 64]
    m = jnp.max(logits, axis=0, keepdims=True)
    e = jnp.exp(logits - m)
    probs = e / jnp.sum(e, axis=0, keepdims=True)
    out_ref[...] = jnp.sum(o * probs[..., None], axis=0)


def _combine(parts, p=512):
    r = parts[0].shape[0]
    n = parts[0].shape[1]
    return pl.pallas_call(
        _combine_body,
        grid=(n // p,),
        in_specs=[pl.BlockSpec((r, p, 128), lambda i: (0, i, 0))
                  for _ in parts],
        out_specs=pl.BlockSpec((p, 64), lambda i: (i, 0)),
        out_shape=jax.ShapeDtypeStruct((n, 64), jnp.float32),
    )(*parts)


def _lsh_hashes(q, k):
    bs, t, dim = q.shape
    qs = jax.lax.stop_gradient(q)
    ks = jax.lax.stop_gradient(k)
    q_norm_sq = jnp.sum(qs * qs, axis=-1, keepdims=True)
    k_norm_sq = jnp.sum(ks * ks, axis=-1, keepdims=True)
    q_max_sq = jnp.max(q_norm_sq, axis=1, keepdims=True)
    k_max_sq = jnp.max(k_norm_sq, axis=1, keepdims=True)
    q_ext = jnp.sqrt(jnp.maximum(q_max_sq - q_norm_sq, 0.0))
    k_ext = jnp.sqrt(jnp.maximum(k_max_sq - k_norm_sq, 0.0))
    Queries = jnp.concatenate([qs, q_ext, jnp.zeros_like(q_ext)], axis=-1)
    Keys = jnp.concatenate([ks, jnp.zeros_like(k_ext), k_ext], axis=-1)
    lkey = jax.random.key(42)
    alpha = jax.random.normal(
        jax.random.fold_in(lkey, 0), (dim + 2, N_HASHES), dtype=jnp.float32)
    beta = jax.random.uniform(
        jax.random.fold_in(lkey, 1), (N_HASHES,), minval=0.0, maxval=R,
        dtype=jnp.float32)
    q_hash = jnp.transpose(Queries @ alpha + beta, (2, 0, 1))
    k_hash = jnp.transpose(Keys @ alpha + beta, (2, 0, 1))
    return q_hash, k_hash


def kernel(query, key, value):
    b, t, h, e = query.shape
    bs = b * h
    q = jnp.transpose(query, (0, 2, 1, 3)).reshape(bs, t, e)
    k = jnp.transpose(key, (0, 2, 1, 3)).reshape(bs, t, e)
    v = jnp.transpose(value, (0, 2, 1, 3)).reshape(bs, t, e)

    q_hash, k_hash = _lsh_hashes(q, k)

    ch = 4
    r = N_HASHES // ch

    offset = (jnp.arange(bs, dtype=jnp.int32) * t)[None, :, None]
    offset2 = (jnp.arange(r * bs, dtype=jnp.int32) * t)[:, None]

    q_tab = jnp.pad(q.reshape(-1, e), ((0, 0), (0, 128 - e)))
    kv_tab = jnp.concatenate(
        [k.reshape(-1, e), v.reshape(-1, e)], axis=1)
    qkv_tab = jnp.concatenate([q_tab, kv_tab], axis=0)

    q_pos_a = jnp.argsort(q_hash, axis=-1).astype(jnp.int32)
    k_pos_a = jnp.argsort(k_hash, axis=-1).astype(jnp.int32)
    iota_t = jnp.broadcast_to(
        jnp.arange(t, dtype=jnp.int32)[None, None], (N_HASHES, bs, t))
    q_rev_a = jnp.zeros((N_HASHES, bs, t), jnp.int32).at[
        jnp.arange(N_HASHES, dtype=jnp.int32)[:, None, None],
        jnp.arange(bs, dtype=jnp.int32)[None, :, None],
        q_pos_a].set(iota_t, mode="promise_in_bounds", unique_indices=True)

    parts = []
    for c in range(ch):
        sl = slice(c * r, (c + 1) * r)
        q_pos = q_pos_a[sl]
        k_pos = k_pos_a[sl]
        q_rev = q_rev_a[sl]
        q_flat = (q_pos + offset).reshape(-1)
        k_flat = (k_pos + offset + bs * t).reshape(-1)
        idx_fwd = jnp.concatenate([q_flat, k_flat])
        q_rev_flat = (q_rev.reshape(r * bs, t) + offset2).reshape(-1)

        s_all = _sc_gather(qkv_tab, idx_fwd)
        bo = _block_attention(s_all.reshape(-1, BLK, 128))
        o_aug = _sc_gather(bo.reshape(-1, 128), q_rev_flat)
        parts.append(o_aug.reshape(r, bs * t, 128))

    out = _combine(parts)
    out = jnp.transpose(out.reshape(b, h, t, e), (0, 2, 1, 3))
    return out


# speedup vs baseline: 1.9866x; 1.9866x over previous
import functools

import jax
import jax.numpy as jnp
from jax import lax
from jax.experimental import pallas as pl
from jax.experimental.pallas import tpu as pltpu
from jax.experimental.pallas import tpu_sc as plsc

N_HASHES = 8
BLK = 128
R = 1.0

_NC = 2
_NS = 16
_NW = _NC * _NS


def _sc_gather_body(per_w, chunk, table_hbm, idx2_hbm, out_hbm,
                    idx_all, rows0, rows1, sw0, sw1, sg0, sg1):
    wid = lax.axis_index("s") * _NC + lax.axis_index("c")
    base = wid * per_w
    nst = per_w // chunk
    pltpu.sync_copy(idx2_hbm.at[pl.ds(base, per_w)], idx_all)

    def step(i, carry):
        off0 = base + (2 * i) * chunk
        off1 = off0 + chunk

        @pl.when(i > 0)
        def _():
            pltpu.make_async_copy(
                rows0, out_hbm.at[pl.ds(base, chunk)], sw0).wait()

        pltpu.async_copy(
            table_hbm.at[idx_all.at[pl.ds((2 * i) * chunk, chunk)]],
            rows0, sg0).wait()
        pltpu.async_copy(rows0, out_hbm.at[pl.ds(off0, chunk)], sw0)

        @pl.when(i > 0)
        def _():
            pltpu.make_async_copy(
                rows1, out_hbm.at[pl.ds(base, chunk)], sw1).wait()

        pltpu.async_copy(
            table_hbm.at[idx_all.at[pl.ds((2 * i + 1) * chunk, chunk)]],
            rows1, sg1).wait()
        pltpu.async_copy(rows1, out_hbm.at[pl.ds(off1, chunk)], sw1)
        return carry

    lax.fori_loop(0, nst // 2, step, 0)
    pltpu.make_async_copy(rows0, out_hbm.at[pl.ds(base, chunk)], sw0).wait()
    pltpu.make_async_copy(rows1, out_hbm.at[pl.ds(base, chunk)], sw1).wait()


def _sc_gather(table, idx, chunk=256):
    n = idx.shape[0]
    d = table.shape[1]
    per_w = n // _NW
    mesh = plsc.VectorSubcoreMesh(core_axis_name="c", subcore_axis_name="s")
    f = pl.kernel(
        functools.partial(_sc_gather_body, per_w, chunk),
        out_type=jax.ShapeDtypeStruct((n, d), table.dtype),
        mesh=mesh,
        scratch_types=[
            pltpu.VMEM((per_w,), jnp.int32),
            pltpu.VMEM((chunk, d), table.dtype),
            pltpu.VMEM((chunk, d), table.dtype),
            pltpu.SemaphoreType.DMA,
            pltpu.SemaphoreType.DMA,
            pltpu.SemaphoreType.DMA,
            pltpu.SemaphoreType.DMA,
        ],
    )
    return f(table, idx)


def _attn_body(q_ref, kv_ref, o_ref):
    q = q_ref[:, :, :64]
    k = kv_ref[:, :, :64]
    v = kv_ref[:, :, 64:]
    inner = jax.lax.dot_general(
        q, k, (((2,), (2,)), ((0,), (0,))), preferred_element_type=jnp.float32)
    m = jnp.max(inner, axis=-1, keepdims=True)
    e = jnp.exp(inner - m)
    s = jnp.sum(e, axis=-1, keepdims=True)
    o = jax.lax.dot_general(
        e, v, (((2,), (1,)), ((0,), (0,))), preferred_element_type=jnp.float32)
    lse = jnp.log(s) + m
    o_ref[...] = jnp.concatenate(
        [o / s, jnp.broadcast_to(lse, o.shape)], axis=-1)


def _block_attention(s_all, g=8):
    nb = s_all.shape[0] // 2
    kv_off = nb // g
    return pl.pallas_call(
        _attn_body,
        grid=(nb // g,),
        in_specs=[
            pl.BlockSpec((g, BLK, 128), lambda i: (i, 0, 0)),
            pl.BlockSpec((g, BLK, 128), lambda i: (i + kv_off, 0, 0)),
        ],
        out_specs=pl.BlockSpec((g, BLK, 128), lambda i: (i, 0, 0)),
        out_shape=jax.ShapeDtypeStruct((nb, BLK, 128), jnp.float32),
    )(s_all, s_all)


def _combine_body(*refs):
    out_ref = refs[-1]
    oa = jnp.concatenate([r[...] for r in refs[:-1]], axis=0)
    o = oa[:, :, :64]
    logits = oa[:, :, 64]
    m = jnp.max(logits, axis=0, keepdims=True)
    e = jnp.exp(logits - m)
    probs = e / jnp.sum(e, axis=0, keepdims=True)
    out_ref[...] = jnp.sum(o * probs[..., None], axis=0)


def _combine(parts, p=512):
    r = parts[0].shape[0]
    n = parts[0].shape[1]
    return pl.pallas_call(
        _combine_body,
        grid=(n // p,),
        in_specs=[pl.BlockSpec((r, p, 128), lambda i: (0, i, 0))
                  for _ in parts],
        out_specs=pl.BlockSpec((p, 64), lambda i: (i, 0)),
        out_shape=jax.ShapeDtypeStruct((n, 64), jnp.float32),
    )(*parts)


def _lsh_hashes(q, k):
    bs, t, dim = q.shape
    qs = jax.lax.stop_gradient(q)
    ks = jax.lax.stop_gradient(k)
    q_norm_sq = jnp.sum(qs * qs, axis=-1, keepdims=True)
    k_norm_sq = jnp.sum(ks * ks, axis=-1, keepdims=True)
    q_max_sq = jnp.max(q_norm_sq, axis=1, keepdims=True)
    k_max_sq = jnp.max(k_norm_sq, axis=1, keepdims=True)
    q_ext = jnp.sqrt(jnp.maximum(q_max_sq - q_norm_sq, 0.0))
    k_ext = jnp.sqrt(jnp.maximum(k_max_sq - k_norm_sq, 0.0))
    Queries = jnp.concatenate([qs, q_ext, jnp.zeros_like(q_ext)], axis=-1)
    Keys = jnp.concatenate([ks, jnp.zeros_like(k_ext), k_ext], axis=-1)
    lkey = jax.random.key(42)
    alpha = jax.random.normal(
        jax.random.fold_in(lkey, 0), (dim + 2, N_HASHES), dtype=jnp.float32)
    beta = jax.random.uniform(
        jax.random.fold_in(lkey, 1), (N_HASHES,), minval=0.0, maxval=R,
        dtype=jnp.float32)
    q_hash = jnp.transpose(Queries @ alpha + beta, (2, 0, 1))
    k_hash = jnp.transpose(Keys @ alpha + beta, (2, 0, 1))
    return q_hash, k_hash


def kernel(query, key, value):
    b, t, h, e = query.shape
    bs = b * h
    q = jnp.transpose(query, (0, 2, 1, 3)).reshape(bs, t, e)
    k = jnp.transpose(key, (0, 2, 1, 3)).reshape(bs, t, e)
    v = jnp.transpose(value, (0, 2, 1, 3)).reshape(bs, t, e)

    q_hash, k_hash = _lsh_hashes(q, k)

    ch = 4
    r = N_HASHES // ch

    offset = (jnp.arange(bs, dtype=jnp.int32) * t)[None, :, None]
    offset2 = (jnp.arange(r * bs, dtype=jnp.int32) * t)[:, None]

    q_tab = jnp.pad(q.reshape(-1, e), ((0, 0), (0, 128 - e)))
    kv_tab = jnp.concatenate(
        [k.reshape(-1, e), v.reshape(-1, e)], axis=1)
    qkv_tab = jnp.concatenate([q_tab, kv_tab], axis=0)

    qk_pos = jnp.argsort(
        jnp.concatenate([q_hash, k_hash], axis=0), axis=-1).astype(jnp.int32)
    q_pos_a, k_pos_a = qk_pos[:N_HASHES], qk_pos[N_HASHES:]
    q_rev_a = jnp.argsort(q_pos_a, axis=-1).astype(jnp.int32)

    parts = []
    for c in range(ch):
        sl = slice(c * r, (c + 1) * r)
        q_pos = q_pos_a[sl]
        k_pos = k_pos_a[sl]
        q_rev = q_rev_a[sl]
        q_flat = (q_pos + offset).reshape(-1)
        k_flat = (k_pos + offset + bs * t).reshape(-1)
        idx_fwd = jnp.concatenate([q_flat, k_flat])
        q_rev_flat = (q_rev.reshape(r * bs, t) + offset2).reshape(-1)

        s_all = _sc_gather(qkv_tab, idx_fwd)
        bo = _block_attention(s_all.reshape(-1, BLK, 128))
        o_aug = _sc_gather(bo.reshape(-1, 128), q_rev_flat)
        parts.append(o_aug.reshape(r, bs * t, 128))

    out = _combine(parts)
    out = jnp.transpose(out.reshape(b, h, t, e), (0, 2, 1, 3))
    return out


# speedup vs baseline: 2.1328x; 1.0736x over previous
import functools

import jax
import jax.numpy as jnp
from jax import lax
from jax.experimental import pallas as pl
from jax.experimental.pallas import tpu as pltpu
from jax.experimental.pallas import tpu_sc as plsc

N_HASHES = 8
BLK = 128
R = 1.0

_NC = 2
_NS = 16
_NW = _NC * _NS


def _sc_gather_body(per_w, chunk, table_hbm, idx2_hbm, out_hbm,
                    idx_all, rows0, rows1, sw0, sw1, sg0, sg1):
    wid = lax.axis_index("s") * _NC + lax.axis_index("c")
    base = wid * per_w
    nst = per_w // chunk
    pltpu.sync_copy(idx2_hbm.at[pl.ds(base, per_w)], idx_all)

    def step(i, carry):
        off0 = base + (2 * i) * chunk
        off1 = off0 + chunk

        @pl.when(i > 0)
        def _():
            pltpu.make_async_copy(
                rows0, out_hbm.at[pl.ds(base, chunk)], sw0).wait()

        pltpu.async_copy(
            table_hbm.at[idx_all.at[pl.ds((2 * i) * chunk, chunk)]],
            rows0, sg0).wait()
        pltpu.async_copy(rows0, out_hbm.at[pl.ds(off0, chunk)], sw0)

        @pl.when(i > 0)
        def _():
            pltpu.make_async_copy(
                rows1, out_hbm.at[pl.ds(base, chunk)], sw1).wait()

        pltpu.async_copy(
            table_hbm.at[idx_all.at[pl.ds((2 * i + 1) * chunk, chunk)]],
            rows1, sg1).wait()
        pltpu.async_copy(rows1, out_hbm.at[pl.ds(off1, chunk)], sw1)
        return carry

    lax.fori_loop(0, nst // 2, step, 0)
    pltpu.make_async_copy(rows0, out_hbm.at[pl.ds(base, chunk)], sw0).wait()
    pltpu.make_async_copy(rows1, out_hbm.at[pl.ds(base, chunk)], sw1).wait()


def _sc_gather(table, idx, chunk=256):
    n = idx.shape[0]
    d = table.shape[1]
    per_w = n // _NW
    mesh = plsc.VectorSubcoreMesh(core_axis_name="c", subcore_axis_name="s")
    f = pl.kernel(
        functools.partial(_sc_gather_body, per_w, chunk),
        out_type=jax.ShapeDtypeStruct((n, d), table.dtype),
        mesh=mesh,
        scratch_types=[
            pltpu.VMEM((per_w,), jnp.int32),
            pltpu.VMEM((chunk, d), table.dtype),
            pltpu.VMEM((chunk, d), table.dtype),
            pltpu.SemaphoreType.DMA,
            pltpu.SemaphoreType.DMA,
            pltpu.SemaphoreType.DMA,
            pltpu.SemaphoreType.DMA,
        ],
    )
    return f(table, idx)


def _attn_body(q_ref, kv_ref, o_ref):
    q = q_ref[:, :, :64]
    k = kv_ref[:, :, :64]
    v = kv_ref[:, :, 64:]
    inner = jax.lax.dot_general(
        q, k, (((2,), (2,)), ((0,), (0,))), preferred_element_type=jnp.float32)
    m = jnp.max(inner, axis=-1, keepdims=True)
    e = jnp.exp(inner - m)
    s = jnp.sum(e, axis=-1, keepdims=True)
    o = jax.lax.dot_general(
        e, v, (((2,), (1,)), ((0,), (0,))), preferred_element_type=jnp.float32)
    lse = jnp.log(s) + m
    o_ref[...] = jnp.concatenate(
        [o / s, jnp.broadcast_to(lse, o.shape)], axis=-1)


def _block_attention(s_all, g=16):
    nb = s_all.shape[0] // 2
    kv_off = nb // g
    return pl.pallas_call(
        _attn_body,
        grid=(nb // g,),
        in_specs=[
            pl.BlockSpec((g, BLK, 128), lambda i: (i, 0, 0)),
            pl.BlockSpec((g, BLK, 128), lambda i: (i + kv_off, 0, 0)),
        ],
        out_specs=pl.BlockSpec((g, BLK, 128), lambda i: (i, 0, 0)),
        out_shape=jax.ShapeDtypeStruct((nb, BLK, 128), jnp.float32),
    )(s_all, s_all)


def _combine_body(*refs):
    out_ref = refs[-1]
    oa = jnp.concatenate([r[...] for r in refs[:-1]], axis=0)
    o = oa[:, :, :64]
    logits = oa[:, :, 64]
    m = jnp.max(logits, axis=0, keepdims=True)
    e = jnp.exp(logits - m)
    probs = e / jnp.sum(e, axis=0, keepdims=True)
    out_ref[...] = jnp.sum(o * probs[..., None], axis=0)


def _combine(parts, p=512):
    r = parts[0].shape[0]
    n = parts[0].shape[1]
    return pl.pallas_call(
        _combine_body,
        grid=(n // p,),
        in_specs=[pl.BlockSpec((r, p, 128), lambda i: (0, i, 0))
                  for _ in parts],
        out_specs=pl.BlockSpec((p, 64), lambda i: (i, 0)),
        out_shape=jax.ShapeDtypeStruct((n, 64), jnp.float32),
    )(*parts)


def _lsh_hashes(q, k):
    bs, t, dim = q.shape
    qs = jax.lax.stop_gradient(q)
    ks = jax.lax.stop_gradient(k)
    q_norm_sq = jnp.sum(qs * qs, axis=-1, keepdims=True)
    k_norm_sq = jnp.sum(ks * ks, axis=-1, keepdims=True)
    q_max_sq = jnp.max(q_norm_sq, axis=1, keepdims=True)
    k_max_sq = jnp.max(k_norm_sq, axis=1, keepdims=True)
    q_ext = jnp.sqrt(jnp.maximum(q_max_sq - q_norm_sq, 0.0))
    k_ext = jnp.sqrt(jnp.maximum(k_max_sq - k_norm_sq, 0.0))
    Queries = jnp.concatenate([qs, q_ext, jnp.zeros_like(q_ext)], axis=-1)
    Keys = jnp.concatenate([ks, jnp.zeros_like(k_ext), k_ext], axis=-1)
    lkey = jax.random.key(42)
    alpha = jax.random.normal(
        jax.random.fold_in(lkey, 0), (dim + 2, N_HASHES), dtype=jnp.float32)
    beta = jax.random.uniform(
        jax.random.fold_in(lkey, 1), (N_HASHES,), minval=0.0, maxval=R,
        dtype=jnp.float32)
    q_hash = jnp.transpose(Queries @ alpha + beta, (2, 0, 1))
    k_hash = jnp.transpose(Keys @ alpha + beta, (2, 0, 1))
    return q_hash, k_hash


def kernel(query, key, value):
    b, t, h, e = query.shape
    bs = b * h
    q = jnp.transpose(query, (0, 2, 1, 3)).reshape(bs, t, e)
    k = jnp.transpose(key, (0, 2, 1, 3)).reshape(bs, t, e)
    v = jnp.transpose(value, (0, 2, 1, 3)).reshape(bs, t, e)

    q_hash, k_hash = _lsh_hashes(q, k)

    ch = 4
    r = N_HASHES // ch

    offset = (jnp.arange(bs, dtype=jnp.int32) * t)[None, :, None]
    offset2 = (jnp.arange(r * bs, dtype=jnp.int32) * t)[:, None]

    q_tab = jnp.pad(q.reshape(-1, e), ((0, 0), (0, 128 - e)))
    kv_tab = jnp.concatenate(
        [k.reshape(-1, e), v.reshape(-1, e)], axis=1)
    qkv_tab = jnp.concatenate([q_tab, kv_tab], axis=0)

    qk_pos = jnp.argsort(
        jnp.concatenate([q_hash, k_hash], axis=0), axis=-1).astype(jnp.int32)
    q_pos_a, k_pos_a = qk_pos[:N_HASHES], qk_pos[N_HASHES:]
    q_rev_a = jnp.argsort(q_pos_a, axis=-1).astype(jnp.int32)

    parts = []
    for c in range(ch):
        sl = slice(c * r, (c + 1) * r)
        q_pos = q_pos_a[sl]
        k_pos = k_pos_a[sl]
        q_rev = q_rev_a[sl]
        q_flat = (q_pos + offset).reshape(-1)
        k_flat = (k_pos + offset + bs * t).reshape(-1)
        idx_fwd = jnp.concatenate([q_flat, k_flat])
        q_rev_flat = (q_rev.reshape(r * bs, t) + offset2).reshape(-1)

        s_all = _sc_gather(qkv_tab, idx_fwd)
        bo = _block_attention(s_all.reshape(-1, BLK, 128))
        o_aug = _sc_gather(bo.reshape(-1, 128), q_rev_flat)
        parts.append(o_aug.reshape(r, bs * t, 128))

    out = _combine(parts)
    out = jnp.transpose(out.reshape(b, h, t, e), (0, 2, 1, 3))
    return out


# speedup vs baseline: 2.1695x; 1.0172x over previous
import functools

import jax
import jax.numpy as jnp
from jax import lax
from jax.experimental import pallas as pl
from jax.experimental.pallas import tpu as pltpu
from jax.experimental.pallas import tpu_sc as plsc

N_HASHES = 8
BLK = 128
R = 1.0

_NC = 2
_NS = 16
_NW = _NC * _NS


def _sc_gather_body(per_w, chunk, table_hbm, idx2_hbm, out_hbm,
                    idx_all, rows0, rows1, sw0, sw1, sg0, sg1):
    wid = lax.axis_index("s") * _NC + lax.axis_index("c")
    base = wid * per_w
    nst = per_w // chunk
    pltpu.sync_copy(idx2_hbm.at[pl.ds(base, per_w)], idx_all)

    def step(i, carry):
        off0 = base + (2 * i) * chunk
        off1 = off0 + chunk

        @pl.when(i > 0)
        def _():
            pltpu.make_async_copy(
                rows0, out_hbm.at[pl.ds(base, chunk)], sw0).wait()

        pltpu.async_copy(
            table_hbm.at[idx_all.at[pl.ds((2 * i) * chunk, chunk)]],
            rows0, sg0).wait()
        pltpu.async_copy(rows0, out_hbm.at[pl.ds(off0, chunk)], sw0)

        @pl.when(i > 0)
        def _():
            pltpu.make_async_copy(
                rows1, out_hbm.at[pl.ds(base, chunk)], sw1).wait()

        pltpu.async_copy(
            table_hbm.at[idx_all.at[pl.ds((2 * i + 1) * chunk, chunk)]],
            rows1, sg1).wait()
        pltpu.async_copy(rows1, out_hbm.at[pl.ds(off1, chunk)], sw1)
        return carry

    lax.fori_loop(0, nst // 2, step, 0)
    pltpu.make_async_copy(rows0, out_hbm.at[pl.ds(base, chunk)], sw0).wait()
    pltpu.make_async_copy(rows1, out_hbm.at[pl.ds(base, chunk)], sw1).wait()


def _sc_gather(table, idx, chunk=256):
    n = idx.shape[0]
    d = table.shape[1]
    per_w = n // _NW
    mesh = plsc.VectorSubcoreMesh(core_axis_name="c", subcore_axis_name="s")
    f = pl.kernel(
        functools.partial(_sc_gather_body, per_w, chunk),
        out_type=jax.ShapeDtypeStruct((n, d), table.dtype),
        mesh=mesh,
        scratch_types=[
            pltpu.VMEM((per_w,), jnp.int32),
            pltpu.VMEM((chunk, d), table.dtype),
            pltpu.VMEM((chunk, d), table.dtype),
            pltpu.SemaphoreType.DMA,
            pltpu.SemaphoreType.DMA,
            pltpu.SemaphoreType.DMA,
            pltpu.SemaphoreType.DMA,
        ],
    )
    return f(table, idx)


def _attn_body(q_ref, kv_ref, o_ref):
    q = q_ref[:, :, :64]
    k = kv_ref[:, :, :64]
    v = kv_ref[:, :, 64:]
    inner = jax.lax.dot_general(
        q, k, (((2,), (2,)), ((0,), (0,))), preferred_element_type=jnp.float32)
    m = jnp.max(inner, axis=-1, keepdims=True)
    e = jnp.exp(inner - m)
    s = jnp.sum(e, axis=-1, keepdims=True)
    o = jax.lax.dot_general(
        e, v, (((2,), (1,)), ((0,), (0,))), preferred_element_type=jnp.float32)
    lse = jnp.log(s) + m
    o_ref[...] = jnp.concatenate(
        [o / s, jnp.broadcast_to(lse, o.shape)], axis=-1)


def _block_attention(s_all, g=32):
    nb = s_all.shape[0] // 2
    kv_off = nb // g
    return pl.pallas_call(
        _attn_body,
        grid=(nb // g,),
        in_specs=[
            pl.BlockSpec((g, BLK, 128), lambda i: (i, 0, 0)),
            pl.BlockSpec((g, BLK, 128), lambda i: (i + kv_off, 0, 0)),
        ],
        out_specs=pl.BlockSpec((g, BLK, 128), lambda i: (i, 0, 0)),
        out_shape=jax.ShapeDtypeStruct((nb, BLK, 128), jnp.float32),
    )(s_all, s_all)


def _combine_body(*refs):
    out_ref = refs[-1]
    oa = jnp.concatenate([r[...] for r in refs[:-1]], axis=0)
    o = oa[:, :, :64]
    logits = oa[:, :, 64]
    m = jnp.max(logits, axis=0, keepdims=True)
    e = jnp.exp(logits - m)
    probs = e / jnp.sum(e, axis=0, keepdims=True)
    out_ref[...] = jnp.sum(o * probs[..., None], axis=0)


def _combine(parts, p=512):
    r = parts[0].shape[0]
    n = parts[0].shape[1]
    return pl.pallas_call(
        _combine_body,
        grid=(n // p,),
        in_specs=[pl.BlockSpec((r, p, 128), lambda i: (0, i, 0))
                  for _ in parts],
        out_specs=pl.BlockSpec((p, 64), lambda i: (i, 0)),
        out_shape=jax.ShapeDtypeStruct((n, 64), jnp.float32),
    )(*parts)


def _lsh_hashes(q, k):
    bs, t, dim = q.shape
    qs = jax.lax.stop_gradient(q)
    ks = jax.lax.stop_gradient(k)
    q_norm_sq = jnp.sum(qs * qs, axis=-1, keepdims=True)
    k_norm_sq = jnp.sum(ks * ks, axis=-1, keepdims=True)
    q_max_sq = jnp.max(q_norm_sq, axis=1, keepdims=True)
    k_max_sq = jnp.max(k_norm_sq, axis=1, keepdims=True)
    q_ext = jnp.sqrt(jnp.maximum(q_max_sq - q_norm_sq, 0.0))
    k_ext = jnp.sqrt(jnp.maximum(k_max_sq - k_norm_sq, 0.0))
    Queries = jnp.concatenate([qs, q_ext, jnp.zeros_like(q_ext)], axis=-1)
    Keys = jnp.concatenate([ks, jnp.zeros_like(k_ext), k_ext], axis=-1)
    lkey = jax.random.key(42)
    alpha = jax.random.normal(
        jax.random.fold_in(lkey, 0), (dim + 2, N_HASHES), dtype=jnp.float32)
    beta = jax.random.uniform(
        jax.random.fold_in(lkey, 1), (N_HASHES,), minval=0.0, maxval=R,
        dtype=jnp.float32)
    q_hash = jnp.transpose(Queries @ alpha + beta, (2, 0, 1))
    k_hash = jnp.transpose(Keys @ alpha + beta, (2, 0, 1))
    return q_hash, k_hash


def kernel(query, key, value):
    b, t, h, e = query.shape
    bs = b * h
    q = jnp.transpose(query, (0, 2, 1, 3)).reshape(bs, t, e)
    k = jnp.transpose(key, (0, 2, 1, 3)).reshape(bs, t, e)
    v = jnp.transpose(value, (0, 2, 1, 3)).reshape(bs, t, e)

    q_hash, k_hash = _lsh_hashes(q, k)

    ch = 4
    r = N_HASHES // ch

    offset = (jnp.arange(bs, dtype=jnp.int32) * t)[None, :, None]
    offset2 = (jnp.arange(r * bs, dtype=jnp.int32) * t)[:, None]

    q_tab = jnp.pad(q.reshape(-1, e), ((0, 0), (0, 128 - e)))
    kv_tab = jnp.concatenate(
        [k.reshape(-1, e), v.reshape(-1, e)], axis=1)
    qkv_tab = jnp.concatenate([q_tab, kv_tab], axis=0)

    qk_pos = jnp.argsort(
        jnp.concatenate([q_hash, k_hash], axis=0), axis=-1).astype(jnp.int32)
    q_pos_a, k_pos_a = qk_pos[:N_HASHES], qk_pos[N_HASHES:]
    q_rev_a = jnp.argsort(q_pos_a, axis=-1).astype(jnp.int32)

    parts = []
    for c in range(ch):
        sl = slice(c * r, (c + 1) * r)
        q_pos = q_pos_a[sl]
        k_pos = k_pos_a[sl]
        q_rev = q_rev_a[sl]
        q_flat = (q_pos + offset).reshape(-1)
        k_flat = (k_pos + offset + bs * t).reshape(-1)
        idx_fwd = jnp.concatenate([q_flat, k_flat])
        q_rev_flat = (q_rev.reshape(r * bs, t) + offset2).reshape(-1)

        s_all = _sc_gather(qkv_tab, idx_fwd)
        bo = _block_attention(s_all.reshape(-1, BLK, 128))
        o_aug = _sc_gather(bo.reshape(-1, 128), q_rev_flat)
        parts.append(o_aug.reshape(r, bs * t, 128))

    out = _combine(parts)
    out = jnp.transpose(out.reshape(b, h, t, e), (0, 2, 1, 3))
    return out
